# Initial kernel scaffold; baseline (speedup 1.0000x reference)
import jax
import jax.numpy as jnp
from jax.experimental import pallas as pl


def kernel(x_dftb, coords, dst_idx, src_idx, W1, b1, Wb, bb, W2, b2, W3, b3):
    raise NotImplementedError("write your pallas kernel here")


# speedup vs baseline: 29.7835x; 29.7835x over previous
import functools

import jax
import jax.numpy as jnp
import numpy as np
from jax import lax
from jax.experimental import pallas as pl
from jax.experimental.pallas import tpu as pltpu
from jax.experimental.pallas import tpu_sc as plsc

f32 = jnp.float32
i32 = jnp.int32

N = 10000
E = 160000
CW = 144
RW = 144
T = 10240
EPAD = 163840
NW = 32
EW = EPAD // NW
CH = 64
NCH = EW // CH
SCH = 512
NSCH = EW // SCH

_INTERPRET = False

_mesh = plsc.VectorSubcoreMesh(core_axis_name="c", subcore_axis_name="s")

_S3 = float(np.sqrt(3.0).astype(np.float32))
_CENTERS = [float(c) for c in np.linspace(0.0, 6.0, 16, dtype=np.float64)]
_INV2W2 = float(1.0 / (2.0 * (6.0 / 16.0) ** 2))

_SC_PARAMS = pltpu.CompilerParams(
    needs_layout_passes=False, use_tc_tiling_on_sc=False)


@functools.partial(
    pl.kernel,
    out_type=jax.ShapeDtypeStruct((EPAD, 32), f32),
    mesh=_mesh,
    scratch_types=(
        pltpu.VMEM((N,), f32),
        pltpu.VMEM((N,), f32),
        pltpu.VMEM((N,), f32),
        pltpu.VMEM((SCH,), i32),
        pltpu.VMEM((SCH,), i32),
        pltpu.VMEM((SCH, 32), f32),
        pltpu.VMEM((512,), f32),
    ),
    compiler_params=_SC_PARAMS,
    interpret=_INTERPRET,
)
def _phase_a(cx_h, cy_h, cz_h, src_h, dst_h, rec_h,
             cx_v, cy_v, cz_v, src_v, dst_v, rec_v, stage_v):
    c = lax.axis_index("c")
    s = lax.axis_index("s")
    base = (s * 2 + c) * EW
    pltpu.sync_copy(cx_h, cx_v)
    pltpu.sync_copy(cy_h, cy_v)
    pltpu.sync_copy(cz_h, cz_v)
    iota16 = lax.iota(i32, 16)
    zeros16 = jnp.zeros((16,), f32)
    ones16 = jnp.ones((16,), f32)
    for rr in range(9, 16):
        stage_v[pl.ds(16 * rr, 16)] = zeros16

    def chunk(j, _):
        cb = base + j * SCH
        pltpu.sync_copy(src_h.at[pl.ds(cb, SCH)], src_v)
        pltpu.sync_copy(dst_h.at[pl.ds(cb, SCH)], dst_v)

        def grp(g, _):
            o = g * 16
            sv = src_v[pl.ds(o, 16)]
            dv = dst_v[pl.ds(o, 16)]
            px = plsc.load_gather(cx_v, [dv]) - plsc.load_gather(cx_v, [sv])
            py = plsc.load_gather(cy_v, [dv]) - plsc.load_gather(cy_v, [sv])
            pz = plsc.load_gather(cz_v, [dv]) - plsc.load_gather(cz_v, [sv])
            sq = px * px + py * py + pz * pz
            yi = plsc.bitcast(jnp.full((16,), 0x5F3759DF, i32)
                              - lax.shift_right_arithmetic(plsc.bitcast(sq, i32),
                                                           jnp.full((16,), 1, i32)),
                              f32)
            for _u in range(3):
                yi = yi * (1.5 - 0.5 * sq * yi * yi)
            d = jnp.where(sq <= 1e-30, jnp.zeros((16,), f32), sq * yi)
            inv = 1.0 / (d + 1e-8)
            ux, uy, uz = px * inv, py * inv, pz * inv
            stage_v[pl.ds(0, 16)] = ones16
            stage_v[pl.ds(16, 16)] = ux
            stage_v[pl.ds(32, 16)] = uy
            stage_v[pl.ds(48, 16)] = uz
            stage_v[pl.ds(64, 16)] = _S3 * ux * uy
            stage_v[pl.ds(80, 16)] = _S3 * uy * uz
            stage_v[pl.ds(96, 16)] = 0.5 * (3.0 * uz * uz - 1.0)
            stage_v[pl.ds(112, 16)] = _S3 * ux * uz
            stage_v[pl.ds(128, 16)] = 0.5 * _S3 * (ux * ux - uy * uy)
            for cc in range(16):
                t = d - _CENTERS[cc]
                stage_v[pl.ds(256 + 16 * cc, 16)] = jnp.exp(t * t * (-_INV2W2))
            for jj in range(16):
                rec_v[o + jj, pl.ds(0, 16)] = plsc.load_gather(
                    stage_v, [iota16 * 16 + jj])
                rec_v[o + jj, pl.ds(16, 16)] = plsc.load_gather(
                    stage_v, [iota16 * 16 + (256 + jj)])
            return 0

        lax.fori_loop(0, SCH // 16, grp, 0)
        pltpu.sync_copy(rec_v, rec_h.at[pl.ds(cb, SCH)])
        return 0

    lax.fori_loop(0, NSCH, chunk, 0)


@functools.partial(
    pl.kernel,
    out_type=jax.ShapeDtypeStruct((2, T, RW), f32),
    mesh=_mesh,
    scratch_types=(
        pltpu.VMEM_SHARED((T, RW), f32),
        pltpu.VMEM((16, RW), f32),
        pltpu.VMEM((CH,), i32),
        pltpu.VMEM((CH,), i32),
        pltpu.VMEM((CH, RW), f32),
        pltpu.VMEM((CH, RW), f32),
        pltpu.VMEM((CH, RW), f32),
        pltpu.SemaphoreType.DMA,
    ),
    compiler_params=_SC_PARAMS,
    interpret=_INTERPRET,
)
def _refine(xa_h, b_h, src_h, dst_h, p_h,
            acc_sh, zb_v, idxs_v, idxd_v, b_v, x_v, m_v, sem):
    c = lax.axis_index("c")
    s = lax.axis_index("s")
    base = (s * 2 + c) * EW
    zeros16 = jnp.zeros((16,), f32)

    def zrow(i, _):
        for jj in range(RW // 16):
            zb_v[i, pl.ds(jj * 16, 16)] = zeros16
        return 0

    lax.fori_loop(0, 16, zrow, 0)

    def zcp(t, _):
        pltpu.sync_copy(zb_v, acc_sh.at[pl.ds(s * 640 + t * 16, 16)])
        return 0

    lax.fori_loop(0, 40, zcp, 0)
    plsc.subcore_barrier()

    def chunk(j, _):
        cb = base + j * CH
        pltpu.sync_copy(src_h.at[pl.ds(cb, CH)], idxs_v)
        pltpu.sync_copy(dst_h.at[pl.ds(cb, CH)], idxd_v)
        pltpu.sync_copy(b_h.at[pl.ds(cb, CH)], b_v)
        pltpu.async_copy(xa_h.at[idxs_v], x_v, sem).wait()

        def edge(e, _):
            b0 = b_v[e, pl.ds(0, 16)]
            xs0 = x_v[e, pl.ds(0, 16)]
            m_v[e, pl.ds(0, 16)] = xs0 * b0
            for k in range(1, 9):
                m_v[e, pl.ds(16 * k, 16)] = (
                    x_v[e, pl.ds(16 * k, 16)] * b0
                    + xs0 * b_v[e, pl.ds(16 * k, 16)])
            return 0

        lax.fori_loop(0, CH, edge, 0)
        pltpu.sync_copy(m_v, acc_sh.at[idxd_v], add=True)
        return 0

    lax.fori_loop(0, NCH, chunk, 0)
    plsc.subcore_barrier()
    pltpu.sync_copy(acc_sh.at[pl.ds(s * 640, 640)],
                    p_h.at[c, pl.ds(s * 640, 640)])


def _bproj_body(rec_ref, w0_ref, w1_ref, w2_ref, bb_ref,
                b0_ref, b1_ref, b2_ref):
    rec = rec_ref[...]
    rbf = rec[:, 16:32]
    db = jnp.concatenate([rec[:, k:k + 1] * rbf for k in range(9)], axis=1)
    b0_ref[...] = jnp.dot(db, w0_ref[...], preferred_element_type=f32) + bb_ref[0:1]
    b1_ref[...] = jnp.dot(db, w1_ref[...], preferred_element_type=f32) + bb_ref[1:2]
    b2_ref[...] = jnp.dot(db, w2_ref[...], preferred_element_type=f32) + bb_ref[2:3]


def _xa0_body(x_ref, w_ref, b_ref, o_ref):
    o_ref[...] = (jnp.dot(x_ref[...], w_ref[...], preferred_element_type=f32)
                  + b_ref[...])


def _tail_body(x_ref, w1_ref, b1_ref, w2_ref, b2_ref, w3_ref, b3_ref, o_ref):
    x1 = jnp.dot(x_ref[...], w1_ref[...], preferred_element_type=f32) + b1_ref[...]
    y = jnp.dot(x1, w2_ref[...], preferred_element_type=f32) + b2_ref[...]
    gate = jnp.tile((y[:, :16] > 0).astype(f32), (1, 9))
    o_ref[...] = jnp.dot(y * gate, w3_ref[...],
                         preferred_element_type=f32) + b3_ref[...]


def _merge_body(a_ref, p_ref, o_ref):
    o_ref[...] = a_ref[...] + p_ref[0] + p_ref[1]


def _head_body(x_ref, w2_ref, b2_ref, w3_ref, b3_ref, o_ref):
    y = (jnp.dot(x_ref[...], w2_ref[...], preferred_element_type=f32)
         + b2_ref[...])
    gate = jnp.tile((y[:, :16] > 0).astype(f32), (1, 9))
    o_ref[...] = jnp.dot(y * gate, w3_ref[...],
                         preferred_element_type=f32) + b3_ref[...]


def _bigw(w):
    ko = w.shape[-1]
    m = jnp.zeros((CW, 9 * ko), f32)
    for k in range(9):
        l = 0 if k < 1 else (1 if k < 4 else 2)
        m = m.at[16 * k:16 * (k + 1), ko * k:ko * (k + 1)].set(w[l])
    return m


def kernel(x_dftb, coords, dst_idx, src_idx, W1, b1, Wb, bb, W2, b2, W3, b3):
    xf = x_dftb.reshape(E, CW)
    src_p = jnp.concatenate([src_idx.astype(i32), jnp.zeros((EPAD - E,), i32)])
    dst_geo = jnp.concatenate([dst_idx.astype(i32), jnp.zeros((EPAD - E,), i32)])
    dst_sc = jnp.concatenate([dst_idx.astype(i32), jnp.full((EPAD - E,), N, i32)])

    bw1 = _bigw(W1)
    bw2 = _bigw(W2)
    bw3 = jnp.pad(_bigw(W3), ((0, 0), (0, 16 - 9)))
    bwb = [_bigw(Wb[i]) for i in range(3)]
    b1v = jnp.pad(b1, (0, CW - 16)).reshape(1, CW)
    b2v = jnp.pad(b2, (0, CW - 16)).reshape(1, CW)
    b3v = jnp.pad(b3, (0, 15)).reshape(1, 16)
    bbv = jnp.pad(bb, ((0, 0), (0, CW - 16)))

    rec = _phase_a(coords[:, 0], coords[:, 1], coords[:, 2], src_p, dst_geo)

    pblk = 1024
    b_its = pl.pallas_call(
        _bproj_body,
        grid=(EPAD // pblk,),
        in_specs=[pl.BlockSpec((pblk, 32), lambda i: (i, 0)),
                  pl.BlockSpec((CW, CW), lambda i: (0, 0)),
                  pl.BlockSpec((CW, CW), lambda i: (0, 0)),
                  pl.BlockSpec((CW, CW), lambda i: (0, 0)),
                  pl.BlockSpec((3, CW), lambda i: (0, 0))],
        out_specs=[pl.BlockSpec((pblk, CW), lambda i: (i, 0))] * 3,
        out_shape=[jax.ShapeDtypeStruct((EPAD, CW), f32)] * 3,
        interpret=_INTERPRET,
    )(rec, bwb[0], bwb[1], bwb[2], bbv)

    xblk = 640
    xa = pl.pallas_call(
        _xa0_body,
        grid=(T // xblk,),
        in_specs=[pl.BlockSpec((xblk, CW), lambda i: (i, 0)),
                  pl.BlockSpec((CW, CW), lambda i: (0, 0)),
                  pl.BlockSpec((1, CW), lambda i: (0, 0))],
        out_specs=pl.BlockSpec((xblk, RW), lambda i: (i, 0)),
        out_shape=jax.ShapeDtypeStruct((T, RW), f32),
        interpret=_INTERPRET,
    )(xf, bw1, b1v)

    tblk = 1000
    out_tail = pl.pallas_call(
        _tail_body,
        grid=((E - N) // tblk,),
        in_specs=[pl.BlockSpec((tblk, CW), lambda i: (i + N // tblk, 0)),
                  pl.BlockSpec((CW, CW), lambda i: (0, 0)),
                  pl.BlockSpec((1, CW), lambda i: (0, 0)),
                  pl.BlockSpec((CW, CW), lambda i: (0, 0)),
                  pl.BlockSpec((1, CW), lambda i: (0, 0)),
                  pl.BlockSpec((CW, 16), lambda i: (0, 0)),
                  pl.BlockSpec((1, 16), lambda i: (0, 0))],
        out_specs=pl.BlockSpec((tblk, 16), lambda i: (i, 0)),
        out_shape=jax.ShapeDtypeStruct((E - N, 16), f32),
        interpret=_INTERPRET,
    )(xf, bw1, b1v, bw2, b2v, bw3, b3v)

    mblk = 640
    for i in range(3):
        p = _refine(xa, b_its[i], src_p, dst_sc)
        xa = pl.pallas_call(
            _merge_body,
            grid=(T // mblk,),
            in_specs=[pl.BlockSpec((mblk, RW), lambda i: (i, 0)),
                      pl.BlockSpec((2, mblk, RW), lambda i: (0, i, 0))],
            out_specs=pl.BlockSpec((mblk, RW), lambda i: (i, 0)),
            out_shape=jax.ShapeDtypeStruct((T, RW), f32),
            interpret=_INTERPRET,
        )(xa, p)

    hblk = 640
    out_head = pl.pallas_call(
        _head_body,
        grid=(T // hblk,),
        in_specs=[pl.BlockSpec((hblk, RW), lambda i: (i, 0)),
                  pl.BlockSpec((CW, CW), lambda i: (0, 0)),
                  pl.BlockSpec((1, CW), lambda i: (0, 0)),
                  pl.BlockSpec((CW, 16), lambda i: (0, 0)),
                  pl.BlockSpec((1, 16), lambda i: (0, 0))],
        out_specs=pl.BlockSpec((hblk, 16), lambda i: (i, 0)),
        out_shape=jax.ShapeDtypeStruct((T, 16), f32),
        interpret=_INTERPRET,
    )(xa, bw2, b2v, bw3, b3v)

    out = jnp.concatenate([out_head[:N, :9], out_tail[:, :9]], axis=0)
    return out.reshape(E, 1, 9, 1)


# speedup vs baseline: 36.0811x; 1.2114x over previous
import functools

import jax
import jax.numpy as jnp
import numpy as np
from jax import lax
from jax.experimental import pallas as pl
from jax.experimental.pallas import tpu as pltpu
from jax.experimental.pallas import tpu_sc as plsc

f32 = jnp.float32
i32 = jnp.int32

N = 10000
E = 160000
CW = 144
RW = 144
T = 10240
EPAD = 163840
NW = 32
EW = EPAD // NW
CH = 64
NCH = EW // CH
SCH = 512
NSCH = EW // SCH

_INTERPRET = False

_mesh = plsc.VectorSubcoreMesh(core_axis_name="c", subcore_axis_name="s")

_S3 = float(np.sqrt(3.0).astype(np.float32))
_CENTERS = [float(c) for c in np.linspace(0.0, 6.0, 16, dtype=np.float64)]
_INV2W2 = float(1.0 / (2.0 * (6.0 / 16.0) ** 2))

_SC_PARAMS = pltpu.CompilerParams(
    needs_layout_passes=False, use_tc_tiling_on_sc=False)


@functools.partial(
    pl.kernel,
    out_type=jax.ShapeDtypeStruct((EPAD, 32), f32),
    mesh=_mesh,
    scratch_types=(
        pltpu.VMEM((N,), f32),
        pltpu.VMEM((N,), f32),
        pltpu.VMEM((N,), f32),
        pltpu.VMEM((SCH,), i32),
        pltpu.VMEM((SCH,), i32),
        pltpu.VMEM((SCH, 32), f32),
        pltpu.VMEM((512,), f32),
    ),
    compiler_params=_SC_PARAMS,
    interpret=_INTERPRET,
)
def _phase_a(cx_h, cy_h, cz_h, src_h, dst_h, rec_h,
             cx_v, cy_v, cz_v, src_v, dst_v, rec_v, stage_v):
    c = lax.axis_index("c")
    s = lax.axis_index("s")
    base = (s * 2 + c) * EW
    pltpu.sync_copy(cx_h, cx_v)
    pltpu.sync_copy(cy_h, cy_v)
    pltpu.sync_copy(cz_h, cz_v)
    iota16 = lax.iota(i32, 16)
    zeros16 = jnp.zeros((16,), f32)
    ones16 = jnp.ones((16,), f32)
    for rr in range(9, 16):
        stage_v[pl.ds(16 * rr, 16)] = zeros16

    def chunk(j, _):
        cb = base + j * SCH
        pltpu.sync_copy(src_h.at[pl.ds(cb, SCH)], src_v)
        pltpu.sync_copy(dst_h.at[pl.ds(cb, SCH)], dst_v)

        def grp(g, _):
            o = g * 16
            sv = src_v[pl.ds(o, 16)]
            dv = dst_v[pl.ds(o, 16)]
            px = plsc.load_gather(cx_v, [dv]) - plsc.load_gather(cx_v, [sv])
            py = plsc.load_gather(cy_v, [dv]) - plsc.load_gather(cy_v, [sv])
            pz = plsc.load_gather(cz_v, [dv]) - plsc.load_gather(cz_v, [sv])
            sq = px * px + py * py + pz * pz
            yi = plsc.bitcast(jnp.full((16,), 0x5F3759DF, i32)
                              - lax.shift_right_arithmetic(plsc.bitcast(sq, i32),
                                                           jnp.full((16,), 1, i32)),
                              f32)
            for _u in range(3):
                yi = yi * (1.5 - 0.5 * sq * yi * yi)
            d = jnp.where(sq <= 1e-30, jnp.zeros((16,), f32), sq * yi)
            inv = 1.0 / (d + 1e-8)
            ux, uy, uz = px * inv, py * inv, pz * inv
            stage_v[pl.ds(0, 16)] = ones16
            stage_v[pl.ds(16, 16)] = ux
            stage_v[pl.ds(32, 16)] = uy
            stage_v[pl.ds(48, 16)] = uz
            stage_v[pl.ds(64, 16)] = _S3 * ux * uy
            stage_v[pl.ds(80, 16)] = _S3 * uy * uz
            stage_v[pl.ds(96, 16)] = 0.5 * (3.0 * uz * uz - 1.0)
            stage_v[pl.ds(112, 16)] = _S3 * ux * uz
            stage_v[pl.ds(128, 16)] = 0.5 * _S3 * (ux * ux - uy * uy)
            for cc in range(16):
                t = d - _CENTERS[cc]
                stage_v[pl.ds(256 + 16 * cc, 16)] = jnp.exp(t * t * (-_INV2W2))
            for jj in range(16):
                rec_v[o + jj, pl.ds(0, 16)] = plsc.load_gather(
                    stage_v, [iota16 * 16 + jj])
                rec_v[o + jj, pl.ds(16, 16)] = plsc.load_gather(
                    stage_v, [iota16 * 16 + (256 + jj)])
            return 0

        lax.fori_loop(0, SCH // 16, grp, 0)
        pltpu.sync_copy(rec_v, rec_h.at[pl.ds(cb, SCH)])
        return 0

    lax.fori_loop(0, NSCH, chunk, 0)


@functools.partial(
    pl.kernel,
    out_type=jax.ShapeDtypeStruct((2, T, RW), f32),
    mesh=_mesh,
    scratch_types=(
        pltpu.VMEM_SHARED((T, RW), f32),
        pltpu.VMEM((8, RW), f32),
        pltpu.VMEM((CH,), i32),
        pltpu.VMEM((CH,), i32),
        pltpu.VMEM((CH,), i32),
        pltpu.VMEM((CH,), i32),
        pltpu.VMEM((CH, RW), f32),
        pltpu.VMEM((CH, RW), f32),
        pltpu.VMEM((CH, RW), f32),
        pltpu.VMEM((CH, RW), f32),
        pltpu.SemaphoreType.DMA,
        pltpu.SemaphoreType.DMA,
        pltpu.SemaphoreType.DMA,
        pltpu.SemaphoreType.DMA,
    ),
    compiler_params=_SC_PARAMS,
    interpret=_INTERPRET,
)
def _refine(xa_h, b_h, src_h, dst_h, p_h,
            acc_sh, zb_v, ixs0, ixs1, ixd0, ixd1, b0_v, b1_v, x0_v, x1_v,
            sg0, sg1, si0, si1):
    c = lax.axis_index("c")
    s = lax.axis_index("s")
    base = (s * 2 + c) * EW
    ixs = [ixs0, ixs1]
    ixd = [ixd0, ixd1]
    bv = [b0_v, b1_v]
    xv = [x0_v, x1_v]
    sg = [sg0, sg1]
    si = [si0, si1]
    zeros16 = jnp.zeros((16,), f32)

    def zrow(i, _):
        for jj in range(RW // 16):
            zb_v[i, pl.ds(jj * 16, 16)] = zeros16
        return 0

    lax.fori_loop(0, 8, zrow, 0)

    def zcp(t, _):
        pltpu.sync_copy(zb_v, acc_sh.at[pl.ds(s * 640 + t * 8, 8)])
        return 0

    lax.fori_loop(0, 80, zcp, 0)

    def issue_main(j, t):
        cb = base + j * CH
        pltpu.async_copy(xa_h.at[ixs[t]], xv[t], sg[t])
        pltpu.async_copy(b_h.at[pl.ds(cb, CH)], bv[t], sg[t])
        pltpu.async_copy(dst_h.at[pl.ds(cb, CH)], ixd[t], sg[t])

    def wait_main(t):
        pltpu.make_async_copy(xa_h.at[ixs[t]], xv[t], sg[t]).wait()
        pltpu.make_async_copy(b_h.at[pl.ds(0, CH)], bv[t], sg[t]).wait()
        pltpu.make_async_copy(dst_h.at[pl.ds(0, CH)], ixd[t], sg[t]).wait()

    pltpu.sync_copy(src_h.at[pl.ds(base, CH)], ixs[0])
    issue_main(0, 0)
    pltpu.async_copy(src_h.at[pl.ds(base + CH, CH)], ixs[1], si[1])
    plsc.subcore_barrier()

    def outer(jj, _):
        for t in range(2):
            j = jj * 2 + t
            wait_main(t)

            @pl.when(j + 1 < NCH)
            def _issue_next():
                pltpu.make_async_copy(
                    src_h.at[pl.ds(0, CH)], ixs[1 - t], si[1 - t]).wait()
                issue_main(j + 1, 1 - t)

            @pl.when(j + 2 < NCH)
            def _prefetch_idx():
                pltpu.async_copy(
                    src_h.at[pl.ds(base + (j + 2) * CH, CH)], ixs[t], si[t])

            def edge(e, _):
                b0 = bv[t][e, pl.ds(0, 16)]
                xs0 = xv[t][e, pl.ds(0, 16)]
                xv[t][e, pl.ds(0, 16)] = xs0 * b0
                for k in range(1, 9):
                    xv[t][e, pl.ds(16 * k, 16)] = (
                        xv[t][e, pl.ds(16 * k, 16)] * b0
                        + xs0 * bv[t][e, pl.ds(16 * k, 16)])
                return 0

            lax.fori_loop(0, CH, edge, 0)
            pltpu.sync_copy(xv[t], acc_sh.at[ixd[t]], add=True)
        return 0

    lax.fori_loop(0, NCH // 2, outer, 0)
    plsc.subcore_barrier()
    pltpu.sync_copy(acc_sh.at[pl.ds(s * 640, 640)],
                    p_h.at[c, pl.ds(s * 640, 640)])


def _bproj_body(rec_ref, w0_ref, w1_ref, w2_ref, bb_ref,
                b0_ref, b1_ref, b2_ref):
    rec = rec_ref[...]
    rbf = rec[:, 16:32]
    db = jnp.concatenate([rec[:, k:k + 1] * rbf for k in range(9)], axis=1)
    b0_ref[...] = jnp.dot(db, w0_ref[...], preferred_element_type=f32) + bb_ref[0:1]
    b1_ref[...] = jnp.dot(db, w1_ref[...], preferred_element_type=f32) + bb_ref[1:2]
    b2_ref[...] = jnp.dot(db, w2_ref[...], preferred_element_type=f32) + bb_ref[2:3]


def _xa0_body(x_ref, w_ref, b_ref, o_ref):
    o_ref[...] = (jnp.dot(x_ref[...], w_ref[...], preferred_element_type=f32)
                  + b_ref[...])


def _tail_body(x_ref, w1_ref, b1_ref, w2_ref, b2_ref, w3_ref, b3_ref, o_ref):
    x1 = jnp.dot(x_ref[...], w1_ref[...], preferred_element_type=f32) + b1_ref[...]
    y = jnp.dot(x1, w2_ref[...], preferred_element_type=f32) + b2_ref[...]
    gate = jnp.tile((y[:, :16] > 0).astype(f32), (1, 9))
    o_ref[...] = jnp.dot(y * gate, w3_ref[...],
                         preferred_element_type=f32) + b3_ref[...]


def _merge_body(a_ref, p_ref, o_ref):
    o_ref[...] = a_ref[...] + p_ref[0] + p_ref[1]


def _head_body(x_ref, w2_ref, b2_ref, w3_ref, b3_ref, o_ref):
    y = (jnp.dot(x_ref[...], w2_ref[...], preferred_element_type=f32)
         + b2_ref[...])
    gate = jnp.tile((y[:, :16] > 0).astype(f32), (1, 9))
    o_ref[...] = jnp.dot(y * gate, w3_ref[...],
                         preferred_element_type=f32) + b3_ref[...]


def _bigw(w):
    ko = w.shape[-1]
    m = jnp.zeros((CW, 9 * ko), f32)
    for k in range(9):
        l = 0 if k < 1 else (1 if k < 4 else 2)
        m = m.at[16 * k:16 * (k + 1), ko * k:ko * (k + 1)].set(w[l])
    return m


def kernel(x_dftb, coords, dst_idx, src_idx, W1, b1, Wb, bb, W2, b2, W3, b3):
    xf = x_dftb.reshape(E, CW)
    src_p = jnp.concatenate([src_idx.astype(i32), jnp.zeros((EPAD - E,), i32)])
    dst_geo = jnp.concatenate([dst_idx.astype(i32), jnp.zeros((EPAD - E,), i32)])
    dst_sc = jnp.concatenate([dst_idx.astype(i32), jnp.full((EPAD - E,), N, i32)])

    bw1 = _bigw(W1)
    bw2 = _bigw(W2)
    bw3 = jnp.pad(_bigw(W3), ((0, 0), (0, 16 - 9)))
    bwb = [_bigw(Wb[i]) for i in range(3)]
    b1v = jnp.pad(b1, (0, CW - 16)).reshape(1, CW)
    b2v = jnp.pad(b2, (0, CW - 16)).reshape(1, CW)
    b3v = jnp.pad(b3, (0, 15)).reshape(1, 16)
    bbv = jnp.pad(bb, ((0, 0), (0, CW - 16)))

    rec = _phase_a(coords[:, 0], coords[:, 1], coords[:, 2], src_p, dst_geo)

    pblk = 1024
    b_its = pl.pallas_call(
        _bproj_body,
        grid=(EPAD // pblk,),
        in_specs=[pl.BlockSpec((pblk, 32), lambda i: (i, 0)),
                  pl.BlockSpec((CW, CW), lambda i: (0, 0)),
                  pl.BlockSpec((CW, CW), lambda i: (0, 0)),
                  pl.BlockSpec((CW, CW), lambda i: (0, 0)),
                  pl.BlockSpec((3, CW), lambda i: (0, 0))],
        out_specs=[pl.BlockSpec((pblk, CW), lambda i: (i, 0))] * 3,
        out_shape=[jax.ShapeDtypeStruct((EPAD, CW), f32)] * 3,
        interpret=_INTERPRET,
    )(rec, bwb[0], bwb[1], bwb[2], bbv)

    xblk = 640
    xa = pl.pallas_call(
        _xa0_body,
        grid=(T // xblk,),
        in_specs=[pl.BlockSpec((xblk, CW), lambda i: (i, 0)),
                  pl.BlockSpec((CW, CW), lambda i: (0, 0)),
                  pl.BlockSpec((1, CW), lambda i: (0, 0))],
        out_specs=pl.BlockSpec((xblk, RW), lambda i: (i, 0)),
        out_shape=jax.ShapeDtypeStruct((T, RW), f32),
        interpret=_INTERPRET,
    )(xf, bw1, b1v)

    tblk = 1000
    out_tail = pl.pallas_call(
        _tail_body,
        grid=((E - N) // tblk,),
        in_specs=[pl.BlockSpec((tblk, CW), lambda i: (i + N // tblk, 0)),
                  pl.BlockSpec((CW, CW), lambda i: (0, 0)),
                  pl.BlockSpec((1, CW), lambda i: (0, 0)),
                  pl.BlockSpec((CW, CW), lambda i: (0, 0)),
                  pl.BlockSpec((1, CW), lambda i: (0, 0)),
                  pl.BlockSpec((CW, 16), lambda i: (0, 0)),
                  pl.BlockSpec((1, 16), lambda i: (0, 0))],
        out_specs=pl.BlockSpec((tblk, 16), lambda i: (i, 0)),
        out_shape=jax.ShapeDtypeStruct((E - N, 16), f32),
        interpret=_INTERPRET,
    )(xf, bw1, b1v, bw2, b2v, bw3, b3v)

    mblk = 640
    for i in range(3):
        p = _refine(xa, b_its[i], src_p, dst_sc)
        xa = pl.pallas_call(
            _merge_body,
            grid=(T // mblk,),
            in_specs=[pl.BlockSpec((mblk, RW), lambda i: (i, 0)),
                      pl.BlockSpec((2, mblk, RW), lambda i: (0, i, 0))],
            out_specs=pl.BlockSpec((mblk, RW), lambda i: (i, 0)),
            out_shape=jax.ShapeDtypeStruct((T, RW), f32),
            interpret=_INTERPRET,
        )(xa, p)

    hblk = 640
    out_head = pl.pallas_call(
        _head_body,
        grid=(T // hblk,),
        in_specs=[pl.BlockSpec((hblk, RW), lambda i: (i, 0)),
                  pl.BlockSpec((CW, CW), lambda i: (0, 0)),
                  pl.BlockSpec((1, CW), lambda i: (0, 0)),
                  pl.BlockSpec((CW, 16), lambda i: (0, 0)),
                  pl.BlockSpec((1, 16), lambda i: (0, 0))],
        out_specs=pl.BlockSpec((hblk, 16), lambda i: (i, 0)),
        out_shape=jax.ShapeDtypeStruct((T, 16), f32),
        interpret=_INTERPRET,
    )(xa, bw2, b2v, bw3, b3v)

    out = jnp.concatenate([out_head[:N, :9], out_tail[:, :9]], axis=0)
    return out.reshape(E, 1, 9, 1)
